# baseline (speedup 1.0000x reference)
import functools
import math

import jax
import jax.numpy as jnp
from jax import lax
from jax.experimental import pallas as pl
from jax.experimental.pallas import tpu as pltpu
from jax.experimental.pallas import tpu_sc as plsc

_NUM_CORES = 2
_NUM_SUBCORES = 16
_NW = _NUM_CORES * _NUM_SUBCORES

_BATCH = 4096
_POS = 200
_D = 64
_G = 128
_CB = 2
_CROWS = _CB * _G
_NBB = _BATCH // _G
_NBLOCKS = _POS * _NBB
_NCH = _NBLOCKS // (_NW * _CB)
_PROW = _D * _BATCH
_TROW = 8 * _G * _NBB
_INV_SCALE = 1.0 / math.sqrt(_D)


def _build():
  mesh = plsc.VectorSubcoreMesh(core_axis_name="c", subcore_axis_name="s")

  @functools.partial(
      pl.kernel,
      mesh=mesh,
      out_type=jax.ShapeDtypeStruct((_POS, 2048, _G), jnp.float32),
      compiler_params=pltpu.CompilerParams(
          use_tc_tiling_on_sc=False, needs_layout_passes=False),
      scratch_types=[
          pltpu.VMEM((2, _CROWS), jnp.int32),
          pltpu.VMEM((2, _CROWS, _D), jnp.float32),
          pltpu.VMEM((2, 128, 133), jnp.float32),
          pltpu.SemaphoreType.DMA,
          pltpu.SemaphoreType.DMA,
          pltpu.SemaphoreType.DMA,
          pltpu.SemaphoreType.DMA,
          pltpu.SemaphoreType.DMA,
          pltpu.SemaphoreType.DMA,
      ],
  )
  def embed(idx_hbm, table_hbm, out_hbm, idx_v, rows_v, t_v,
            gsem0, gsem1, ssem0, ssem1, isem0, isem1):
    gsems = (gsem0, gsem1)
    ssems = (ssem0, ssem1)
    isems = (isem0, isem1)
    wid = lax.axis_index("s") * _NUM_CORES + lax.axis_index("c")
    blk0 = wid * _NCH * _CB

    iota = lax.iota(jnp.int32, 16)
    s_idx = lax.bitwise_and(iota, 7)
    hi = lax.shift_right_logical(iota, 3)
    rowv = [[(2 * q + hi) * 16 + cc * 8 + s_idx for q in range(4)]
            for cc in range(_CB)]

    def chunk_pc(k):
      blk = blk0 + k * _CB
      p = lax.shift_right_logical(blk, 5)
      c0 = lax.bitwise_and(blk, 31)
      return p, c0

    def start_idx_load(k, b):
      p, c0 = chunk_pc(k)
      st = p * _BATCH + c0 * _G
      pltpu.async_copy(idx_hbm.at[pl.ds(st, _CROWS)], idx_v.at[b], isems[b])

    def wait_idx_load(b):
      pltpu.make_async_copy(
          idx_hbm.at[pl.ds(0, _CROWS)], idx_v.at[b], isems[b]).wait()

    def start_gathers(b):
      for j in range(_CB):
        pltpu.async_copy(
            table_hbm.at[idx_v.at[b, pl.ds(j * _G, _G)]],
            rows_v.at[b, pl.ds(j * _G, _G)],
            gsems[b])

    def wait_gathers(b):
      pltpu.make_async_copy(
          table_hbm.at[pl.ds(0, _CROWS)], rows_v.at[b], gsems[b]).wait()

    def start_scatters(k, b):
      p, c0 = chunk_pc(k)
      for t in range(8):
        pltpu.async_copy(
            t_v.at[b, pl.ds(t * 16, 16), pl.ds(0, _G)],
            out_hbm.at[p, pl.ds(t * 256 + c0 * 8, 16), :],
            ssems[b])

    def wait_scatters(b):
      pltpu.make_async_copy(
          t_v.at[b, pl.ds(0, 128), pl.ds(0, _G)],
          out_hbm.at[0, pl.ds(0, 128), :], ssems[b]).wait()

    def transpose_scale(b):
      for cc in range(_CB):
        def body(i, carry, cc=cc):
          for u in range(16):
            bb = i * 16 + u
            r = cc * _G + bb
            colv = jnp.full((16,), 0, jnp.int32) + bb
            for q in range(4):
              v = rows_v[b, r, pl.ds(q * 16, 16)] * _INV_SCALE
              plsc.store_scatter(t_v.at[b], [rowv[cc][q], colv], v)
          return carry
        lax.fori_loop(0, _G // 16, body, 0)

    start_idx_load(0, 0)
    wait_idx_load(0)
    start_gathers(0)
    start_idx_load(1, 1)

    def outer(i, carry):
      for b in range(2):
        k = 2 * i + b
        nb = 1 - b

        @pl.when(k + 1 < _NCH)
        def _prefetch():
          @pl.when(k >= 1)
          def _drain_prev_scatter():
            wait_scatters(nb)
          wait_idx_load(nb)
          start_gathers(nb)

        wait_gathers(b)

        @pl.when(k + 2 < _NCH)
        def _next_idx():
          start_idx_load(k + 2, b)

        transpose_scale(b)
        start_scatters(k, b)
      return carry

    lax.fori_loop(0, _NCH // 2, outer, 0)
    wait_scatters(0)
    wait_scatters(1)

  return embed


_EMBED = _build()


def kernel(x, table):
  idx2 = x.T.reshape(_POS * _BATCH).astype(jnp.int32)
  out3 = _EMBED(idx2, table)
  out5 = out3.reshape(_POS, 8, _NBB, 8, _G)
  return out5.transpose(2, 4, 0, 1, 3).reshape(_BATCH, _POS, _D)


# speedup vs baseline: 1.5071x; 1.5071x over previous
import functools
import math

import jax
import jax.numpy as jnp
from jax import lax
from jax.experimental import pallas as pl
from jax.experimental.pallas import tpu as pltpu
from jax.experimental.pallas import tpu_sc as plsc

_NUM_CORES = 2
_NUM_SUBCORES = 16
_NW = _NUM_CORES * _NUM_SUBCORES

_BATCH = 4096
_POS = 200
_D = 64
_G = 128
_CB = 2
_CROWS = _CB * _G
_NBB = _BATCH // _G
_NBLOCKS = _POS * _NBB
_NCH = _NBLOCKS // (_NW * _CB)
_PROW = _D * _BATCH
_TROW = 8 * _G * _NBB
_INV_SCALE = 1.0 / math.sqrt(_D)


def _build():
  mesh = plsc.VectorSubcoreMesh(core_axis_name="c", subcore_axis_name="s")

  @functools.partial(
      pl.kernel,
      mesh=mesh,
      out_type=jax.ShapeDtypeStruct((_POS, 2048, _G), jnp.float32),
      compiler_params=pltpu.CompilerParams(
          use_tc_tiling_on_sc=False, needs_layout_passes=False),
      scratch_types=[
          pltpu.VMEM((2, _CROWS), jnp.int32),
          pltpu.VMEM((2, _CROWS, _D), jnp.float32),
          pltpu.VMEM((2, 128, 133), jnp.float32),
          pltpu.SemaphoreType.DMA,
          pltpu.SemaphoreType.DMA,
          pltpu.SemaphoreType.DMA,
          pltpu.SemaphoreType.DMA,
          pltpu.SemaphoreType.DMA,
          pltpu.SemaphoreType.DMA,
      ],
  )
  def embed(idx_hbm, table_hbm, out_hbm, idx_v, rows_v, t_v,
            gsem0, gsem1, ssem0, ssem1, isem0, isem1):
    gsems = (gsem0, gsem1)
    ssems = (ssem0, ssem1)
    isems = (isem0, isem1)
    wid = lax.axis_index("s") * _NUM_CORES + lax.axis_index("c")
    blk0 = wid * _NCH * _CB

    iota = lax.iota(jnp.int32, 16)
    s_idx = lax.bitwise_and(iota, 7)
    hi = lax.shift_right_logical(iota, 3)
    rowv = [[(2 * q + hi) * 16 + cc * 8 + s_idx for q in range(4)]
            for cc in range(_CB)]

    def chunk_pc(k):
      blk = blk0 + k * _CB
      p = lax.shift_right_logical(blk, 5)
      c0 = lax.bitwise_and(blk, 31)
      return p, c0

    def start_idx_load(k, b):
      p, c0 = chunk_pc(k)
      st = p * _BATCH + c0 * _G
      pltpu.async_copy(idx_hbm.at[pl.ds(st, _CROWS)], idx_v.at[b], isems[b])

    def wait_idx_load(b):
      pltpu.make_async_copy(
          idx_hbm.at[pl.ds(0, _CROWS)], idx_v.at[b], isems[b]).wait()

    def start_gathers(b):
      for j in range(_CB):
        pltpu.async_copy(
            table_hbm.at[idx_v.at[b, pl.ds(j * _G, _G)]],
            rows_v.at[b, pl.ds(j * _G, _G)],
            gsems[b])

    def wait_gathers(b):
      pltpu.make_async_copy(
          table_hbm.at[pl.ds(0, _CROWS)], rows_v.at[b], gsems[b]).wait()

    def start_scatters(k, b):
      p, c0 = chunk_pc(k)
      for t in range(8):
        pltpu.async_copy(
            t_v.at[b, pl.ds(t * 16, 16), pl.ds(0, _G)],
            out_hbm.at[p, pl.ds(t * 256 + c0 * 8, 16), :],
            ssems[b])

    def wait_scatters(b):
      pltpu.make_async_copy(
          t_v.at[b, pl.ds(0, 128), pl.ds(0, _G)],
          out_hbm.at[0, pl.ds(0, 128), :], ssems[b]).wait()

    def transpose_scale(b):
      for cc in range(_CB):
        @plsc.parallel_loop(0, _G, unroll=8)
        def _body(bb, cc=cc):
          r = cc * _G + bb
          colv = jnp.full((16,), 0, jnp.int32) + bb
          for q in range(4):
            v = rows_v[b, r, pl.ds(q * 16, 16)] * _INV_SCALE
            plsc.store_scatter(t_v.at[b], [rowv[cc][q], colv], v)

    start_idx_load(0, 0)
    wait_idx_load(0)
    start_gathers(0)
    start_idx_load(1, 1)

    def outer(i, carry):
      for b in range(2):
        k = 2 * i + b
        nb = 1 - b

        @pl.when(k + 1 < _NCH)
        def _prefetch():
          @pl.when(k >= 1)
          def _drain_prev_scatter():
            wait_scatters(nb)
          wait_idx_load(nb)
          start_gathers(nb)

        wait_gathers(b)

        @pl.when(k + 2 < _NCH)
        def _next_idx():
          start_idx_load(k + 2, b)

        transpose_scale(b)
        start_scatters(k, b)
      return carry

    lax.fori_loop(0, _NCH // 2, outer, 0)
    wait_scatters(0)
    wait_scatters(1)

  return embed


_EMBED = _build()


def kernel(x, table):
  idx2 = x.T.reshape(_POS * _BATCH).astype(jnp.int32)
  out3 = _EMBED(idx2, table)
  out5 = out3.reshape(_POS, 8, _NBB, 8, _G)
  return out5.transpose(2, 4, 0, 1, 3).reshape(_BATCH, _POS, _D)


# speedup vs baseline: 1.5100x; 1.0019x over previous
import functools
import math

import jax
import jax.numpy as jnp
from jax import lax
from jax.experimental import pallas as pl
from jax.experimental.pallas import tpu as pltpu
from jax.experimental.pallas import tpu_sc as plsc

_NUM_CORES = 2
_NUM_SUBCORES = 16
_NW = _NUM_CORES * _NUM_SUBCORES

_BATCH = 4096
_POS = 200
_D = 64
_G = 128
_CB = 2
_CROWS = _CB * _G
_NBB = _BATCH // _G
_NBLOCKS = _POS * _NBB
_NCH = _NBLOCKS // (_NW * _CB)
_PROW = _D * _BATCH
_TROW = 8 * _G * _NBB
_INV_SCALE = 1.0 / math.sqrt(_D)


def _build():
  mesh = plsc.VectorSubcoreMesh(core_axis_name="c", subcore_axis_name="s")

  @functools.partial(
      pl.kernel,
      mesh=mesh,
      out_type=jax.ShapeDtypeStruct((_POS, 8, 256, _G), jnp.float32),
      compiler_params=pltpu.CompilerParams(
          use_tc_tiling_on_sc=False, needs_layout_passes=False),
      scratch_types=[
          pltpu.VMEM((2, _CROWS), jnp.int32),
          pltpu.VMEM((2, _CROWS, _D), jnp.float32),
          pltpu.VMEM((2, 8, 16, 133), jnp.float32),
          pltpu.SemaphoreType.DMA,
          pltpu.SemaphoreType.DMA,
          pltpu.SemaphoreType.DMA,
          pltpu.SemaphoreType.DMA,
          pltpu.SemaphoreType.DMA,
          pltpu.SemaphoreType.DMA,
      ],
  )
  def embed(idx_hbm, table_hbm, out_hbm, idx_v, rows_v, t_v,
            gsem0, gsem1, ssem0, ssem1, isem0, isem1):
    gsems = (gsem0, gsem1)
    ssems = (ssem0, ssem1)
    isems = (isem0, isem1)
    wid = lax.axis_index("s") * _NUM_CORES + lax.axis_index("c")
    blk0 = wid * _NCH * _CB

    iota = lax.iota(jnp.int32, 16)
    s_idx = lax.bitwise_and(iota, 7)
    hi = lax.shift_right_logical(iota, 3)
    t_vec = [2 * q + hi for q in range(4)]
    r2v = [cc * 8 + s_idx for cc in range(_CB)]

    def chunk_pc(k):
      blk = blk0 + k * _CB
      p = lax.shift_right_logical(blk, 5)
      c0 = lax.bitwise_and(blk, 31)
      return p, c0

    def start_idx_load(k, b):
      p, c0 = chunk_pc(k)
      st = p * _BATCH + c0 * _G
      pltpu.async_copy(idx_hbm.at[pl.ds(st, _CROWS)], idx_v.at[b], isems[b])

    def wait_idx_load(b):
      pltpu.make_async_copy(
          idx_hbm.at[pl.ds(0, _CROWS)], idx_v.at[b], isems[b]).wait()

    def start_gathers(b):
      for j in range(_CB):
        pltpu.async_copy(
            table_hbm.at[idx_v.at[b, pl.ds(j * _G, _G)]],
            rows_v.at[b, pl.ds(j * _G, _G)],
            gsems[b])

    def wait_gathers(b):
      pltpu.make_async_copy(
          table_hbm.at[pl.ds(0, _CROWS)], rows_v.at[b], gsems[b]).wait()

    def start_scatters(k, b):
      p, c0 = chunk_pc(k)
      pltpu.async_copy(
          t_v.at[b, :, :, pl.ds(0, _G)],
          out_hbm.at[p, :, pl.ds(c0 * 8, 16), :],
          ssems[b])

    def wait_scatters(b):
      pltpu.make_async_copy(
          t_v.at[b, :, :, pl.ds(0, _G)],
          out_hbm.at[0, :, pl.ds(0, 16), :], ssems[b]).wait()

    def transpose_scale(b):
      for cc in range(_CB):
        @plsc.parallel_loop(0, _G, unroll=8)
        def _body(bb, cc=cc):
          r = cc * _G + bb
          colv = jnp.full((16,), 0, jnp.int32) + bb
          for q in range(4):
            v = rows_v[b, r, pl.ds(q * 16, 16)] * _INV_SCALE
            plsc.store_scatter(t_v.at[b], [t_vec[q], r2v[cc], colv], v)

    start_idx_load(0, 0)
    wait_idx_load(0)
    start_gathers(0)
    start_idx_load(1, 1)

    def outer(i, carry):
      for b in range(2):
        k = 2 * i + b
        nb = 1 - b

        @pl.when(k + 1 < _NCH)
        def _prefetch():
          @pl.when(k >= 1)
          def _drain_prev_scatter():
            wait_scatters(nb)
          wait_idx_load(nb)
          start_gathers(nb)

        wait_gathers(b)

        @pl.when(k + 2 < _NCH)
        def _next_idx():
          start_idx_load(k + 2, b)

        transpose_scale(b)
        start_scatters(k, b)
      return carry

    lax.fori_loop(0, _NCH // 2, outer, 0)
    wait_scatters(0)
    wait_scatters(1)

  return embed


_EMBED = _build()


def kernel(x, table):
  idx2 = x.T.reshape(_POS * _BATCH).astype(jnp.int32)
  out3 = _EMBED(idx2, table)
  out5 = out3.reshape(_POS, 8, _NBB, 8, _G)
  return out5.transpose(2, 4, 0, 1, 3).reshape(_BATCH, _POS, _D)
